# Initial kernel scaffold; baseline (speedup 1.0000x reference)
import jax
import jax.numpy as jnp
from jax.experimental import pallas as pl


def kernel(x, edge_index, batch, hlr, std, W1, b1, W2, b2, W3, b3, Wf1, bf1, Wf2, bf2):
    raise NotImplementedError("write your pallas kernel here")


# speedup vs baseline: 10.7231x; 10.7231x over previous
import functools

import jax
import jax.numpy as jnp
from jax import lax
from jax.experimental import pallas as pl
from jax.experimental.pallas import tpu as pltpu
from jax.experimental.pallas import tpu_sc as plsc

_N = 10000
_E = 320000
_G = 16
_D = 128
_DOUT = 64

_NC = 2
_NS = 16
_NW = _NC * _NS
_K = 80
_NP = 10240

_BLK = 1000
_NB = _N // _BLK


def _make_propagate(d, gather=True):
    ew = _E // _NW
    nchunks = ew // _K
    rps = _NP // _NS
    zr = 128
    mesh = plsc.VectorSubcoreMesh(core_axis_name="c", subcore_axis_name="s")

    @functools.partial(
        pl.kernel, mesh=mesh,
        out_type=jax.ShapeDtypeStruct((2 * _NP, d), jnp.float32),
        scratch_types=[
            pltpu.VMEM((_K,), jnp.int32),
            pltpu.VMEM((_K,), jnp.int32),
            pltpu.VMEM((_K, d), jnp.float32),
            pltpu.VMEM((zr, d), jnp.float32),
            pltpu.VMEM_SHARED((_NP, d), jnp.float32),
            pltpu.SemaphoreType.DMA,
        ],
    )
    def prop(src_hbm, dst_hbm, t_hbm, out_hbm, sidx, didx, rows, zbuf, acc, sem):
        cid = lax.axis_index("c")
        sid = lax.axis_index("s")
        wid = cid * _NS + sid

        zvec = jnp.zeros((16,), jnp.float32)

        def zb_body(i, carry):
            for j in range(d // 16):
                zbuf[i, pl.ds(j * 16, 16)] = zvec
            return carry

        lax.fori_loop(0, zr, zb_body, 0)

        if not gather:
            ovec = jnp.ones((16,), jnp.float32)

            def ones_body(i, carry):
                for j in range(d // 16):
                    rows[i, pl.ds(j * 16, 16)] = ovec
                return carry

            lax.fori_loop(0, _K, ones_body, 0)

        row0 = sid * rps
        for j in range(rps // zr):
            pltpu.sync_copy(zbuf, acc.at[pl.ds(row0 + j * zr, zr)])
        plsc.subcore_barrier()

        base = wid * ew

        def body(i, carry):
            off = base + i * _K
            if gather:
                pltpu.sync_copy(src_hbm.at[pl.ds(off, _K)], sidx)
                pltpu.async_copy(t_hbm.at[sidx], rows, sem).wait()
            pltpu.sync_copy(dst_hbm.at[pl.ds(off, _K)], didx)
            pltpu.sync_copy(rows, acc.at[didx], add=True)
            return carry

        lax.fori_loop(0, nchunks, body, 0)

        plsc.subcore_barrier()
        pltpu.sync_copy(acc.at[pl.ds(row0, rps)],
                        out_hbm.at[pl.ds(cid * _NP + row0, rps)])

    return prop


_prop128 = _make_propagate(_D)
_degprop = _make_propagate(_D, gather=False)


def _tca(p0d, p1d, x, w):
    def body(p0_ref, p1_ref, x_ref, w_ref, t_ref, dis_ref):
        deg = p0_ref[:, 0:1] + p1_ref[:, 0:1] + 1.0
        dis = lax.rsqrt(deg)
        h = jnp.dot(x_ref[...], w_ref[...], preferred_element_type=jnp.float32)
        t_ref[...] = h * dis
        dis_ref[...] = jnp.broadcast_to(dis, h.shape)

    return pl.pallas_call(
        body,
        grid=(_NB,),
        in_specs=[
            pl.BlockSpec((_BLK, _D), lambda i: (i, 0)),
            pl.BlockSpec((_BLK, _D), lambda i: (i, 0)),
            pl.BlockSpec((_BLK, _D), lambda i: (i, 0)),
            pl.BlockSpec((_D, _D), lambda i: (0, 0)),
        ],
        out_specs=[
            pl.BlockSpec((_BLK, _D), lambda i: (i, 0)),
            pl.BlockSpec((_BLK, _D), lambda i: (i, 0)),
        ],
        out_shape=[
            jax.ShapeDtypeStruct((_N, _D), jnp.float32),
            jax.ShapeDtypeStruct((_N, _D), jnp.float32),
        ],
    )(p0d, p1d, x, w)


def _tcb(p0, p1, t, dis, b, w):
    def body(p0_ref, p1_ref, t_ref, dis_ref, b_ref, w_ref, out_ref):
        dis = dis_ref[...]
        h = jnp.maximum((p0_ref[...] + p1_ref[...] + t_ref[...]) * dis
                        + b_ref[...], 0.0)
        out_ref[...] = jnp.dot(h, w_ref[...],
                               preferred_element_type=jnp.float32) * dis

    return pl.pallas_call(
        body,
        grid=(_NB,),
        in_specs=[
            pl.BlockSpec((_BLK, _D), lambda i: (i, 0)),
            pl.BlockSpec((_BLK, _D), lambda i: (i, 0)),
            pl.BlockSpec((_BLK, _D), lambda i: (i, 0)),
            pl.BlockSpec((_BLK, _D), lambda i: (i, 0)),
            pl.BlockSpec((1, _D), lambda i: (0, 0)),
            pl.BlockSpec((_D, _D), lambda i: (0, 0)),
        ],
        out_specs=pl.BlockSpec((_BLK, _D), lambda i: (i, 0)),
        out_shape=jax.ShapeDtypeStruct((_N, _D), jnp.float32),
    )(p0, p1, t, dis, b, w)


def _tcc(p0, p1, t, dis, b, batch3, hlr, std, wf1a, w_hlr, w_std, bf1, wf2, bf2):
    def body(p0_ref, p1_ref, t_ref, dis_ref, b_ref, batch_ref, hlr_ref,
             std_ref, wf1_ref, whlr_ref, wstd_ref, bf1_ref, wf2_ref, bf2_ref,
             out_ref, s_acc, c_acc):
        i = pl.program_id(0)

        @pl.when(i == 0)
        def _():
            s_acc[...] = jnp.zeros_like(s_acc)
            c_acc[...] = jnp.zeros_like(c_acc)

        h = jnp.maximum((p0_ref[...] + p1_ref[...] + t_ref[...])
                        * dis_ref[...] + b_ref[...], 0.0)
        bb = batch_ref[0, 0, :]
        iota = lax.broadcasted_iota(jnp.int32, (_BLK, _G), 1)
        oh = (bb[:, None] == iota).astype(jnp.float32)
        s_acc[...] += lax.dot_general(
            oh, h, dimension_numbers=(((0,), (0,)), ((), ())),
            preferred_element_type=jnp.float32)
        c_acc[...] += jnp.broadcast_to(jnp.sum(oh, axis=0)[:, None],
                                       (_G, _D))

        @pl.when(i == _NB - 1)
        def _():
            g = s_acc[...] / jnp.maximum(c_acc[...], 1.0)
            z = (jnp.dot(g, wf1_ref[...], preferred_element_type=jnp.float32)
                 + hlr_ref[...] * whlr_ref[...]
                 + std_ref[...] * wstd_ref[...] + bf1_ref[...])
            z = jnp.maximum(z, 0.0)
            out_ref[...] = (jnp.dot(z, wf2_ref[...],
                                    preferred_element_type=jnp.float32)
                            + bf2_ref[...])

    return pl.pallas_call(
        body,
        grid=(_NB,),
        in_specs=[
            pl.BlockSpec((_BLK, _D), lambda i: (i, 0)),
            pl.BlockSpec((_BLK, _D), lambda i: (i, 0)),
            pl.BlockSpec((_BLK, _D), lambda i: (i, 0)),
            pl.BlockSpec((_BLK, _D), lambda i: (i, 0)),
            pl.BlockSpec((1, _D), lambda i: (0, 0)),
            pl.BlockSpec((1, 1, _BLK), lambda i: (i, 0, 0)),
            pl.BlockSpec((_G, 1), lambda i: (0, 0)),
            pl.BlockSpec((_G, 1), lambda i: (0, 0)),
            pl.BlockSpec((_D, _D), lambda i: (0, 0)),
            pl.BlockSpec((1, _D), lambda i: (0, 0)),
            pl.BlockSpec((1, _D), lambda i: (0, 0)),
            pl.BlockSpec((1, _D), lambda i: (0, 0)),
            pl.BlockSpec((_D, _DOUT), lambda i: (0, 0)),
            pl.BlockSpec((1, _DOUT), lambda i: (0, 0)),
        ],
        out_specs=pl.BlockSpec((_G, _DOUT), lambda i: (0, 0)),
        out_shape=jax.ShapeDtypeStruct((_G, _DOUT), jnp.float32),
        scratch_shapes=[
            pltpu.VMEM((_G, _D), jnp.float32),
            pltpu.VMEM((_G, _D), jnp.float32),
        ],
    )(p0, p1, t, dis, b, batch3, hlr, std, wf1a, w_hlr, w_std, bf1, wf2, bf2)


def kernel(x, edge_index, batch, hlr, std, W1, b1, W2, b2, W3, b3,
           Wf1, bf1, Wf2, bf2):
    src = edge_index[0]
    dst = edge_index[1]

    pdeg = _degprop(src, dst, x)

    t1, dis = _tca(pdeg[:_N], pdeg[_NP:_NP + _N], x, W1)

    p = _prop128(src, dst, t1)
    t2 = _tcb(p[:_N], p[_NP:_NP + _N], t1, dis, b1.reshape(1, _D), W2)

    p = _prop128(src, dst, t2)
    t3 = _tcb(p[:_N], p[_NP:_NP + _N], t2, dis, b2.reshape(1, _D), W3)

    p = _prop128(src, dst, t3)
    out = _tcc(p[:_N], p[_NP:_NP + _N], t3, dis, b3.reshape(1, _D),
               batch.reshape(_NB, 1, _BLK), hlr, std,
               Wf1[:_D], Wf1[_D:_D + 1], Wf1[_D + 1:_D + 2],
               bf1.reshape(1, _D), Wf2, bf2.reshape(1, _DOUT))
    return out
